# baseline (speedup 1.0000x reference)
import functools

import numpy as np
import jax
import jax.numpy as jnp
from jax import lax
from jax.experimental import pallas as pl
from jax.experimental.pallas import tpu as pltpu
from jax.experimental.pallas import tpu_sc as plsc

_NC = 2
_NS = 16
_NW = _NC * _NS
_G = 8


def _sc_gather_pool(name, collaborative, track_uri_can, seq_flat, tables):
  B = name.shape[0]
  L = seq_flat[0].shape[0] // B
  ED = tables[0].shape[1]
  BW = B // _NW
  NJ = ED // 16
  NCHUNK = BW // _G
  GL = _G * L

  mesh = plsc.VectorSubcoreMesh(
      core_axis_name="c", subcore_axis_name="s",
      num_cores=_NC, num_subcores=_NS)

  out_type = tuple(
      jax.ShapeDtypeStruct((B, ED), jnp.float32) for _ in range(8))

  @functools.partial(
      pl.kernel, mesh=mesh, out_type=out_type,
      scratch_types=[
          pltpu.VMEM((BW,), jnp.int32),
          pltpu.VMEM((BW * L,), jnp.int32),
          pltpu.VMEM((GL, ED), jnp.float32),
          pltpu.VMEM((GL, ED), jnp.float32),
          pltpu.VMEM((BW, ED), jnp.float32),
          pltpu.SemaphoreType.DMA,
          pltpu.SemaphoreType.DMA,
          pltpu.SemaphoreType.DMA,
      ])
  def sc_kernel(name_h, collab_h, can_h, an_h, tu_h, tn_h, al_h, ag_h,
                t_name, t_collab, t_can, t_an, t_tu, t_tn, t_al, t_ag,
                o_name, o_collab, o_can, o_an, o_tu, o_tn, o_al, o_ag,
                idx1_v, sidx_v, buf0, buf1, sums_v,
                sem_a, sem0, sem1):
    rows_v = sums_v
    wid = lax.axis_index("s") * _NC + lax.axis_index("c")
    base = wid * BW

    for idx_h, tab, out in ((name_h, t_name, o_name),
                            (collab_h, t_collab, o_collab),
                            (can_h, t_can, o_can)):
      pltpu.sync_copy(idx_h.at[pl.ds(base, BW)], idx1_v)
      pltpu.async_copy(tab.at[idx1_v], rows_v, sem_a).wait()
      pltpu.sync_copy(rows_v, out.at[pl.ds(base, BW)])

    bufs = (buf0, buf1)
    sems = (sem0, sem1)
    for seq_h, tab, out in ((an_h, t_an, o_an), (tu_h, t_tu, o_tu),
                            (tn_h, t_tn, o_tn), (al_h, t_al, o_al),
                            (ag_h, t_ag, o_ag)):
      pltpu.sync_copy(seq_h.at[pl.ds(base * L, BW * L)], sidx_v)

      def start(ci, k):
        pltpu.async_copy(
            tab.at[sidx_v.at[pl.ds(ci * GL, GL)]], bufs[k], sems[k])

      def wait(k):
        pltpu.make_async_copy(
            tab.at[sidx_v.at[pl.ds(0, GL)]], bufs[k], sems[k]).wait()

      start(0, 0)
      start(1, 1)

      def outer(c2, carry):
        for k in range(2):
          ci = c2 * 2 + k
          wait(k)

          @pl.when(ci + 2 < NCHUNK)
          def _(k=k, ci=ci):
            start(ci + 2, k)

          buf = bufs[k]
          for g in range(_G):
            def red(l, accs, _g=g, _buf=buf):
              row = _g * L + l
              return tuple(accs[j] + _buf[row, pl.ds(j * 16, 16)]
                           for j in range(NJ))
            accs = lax.fori_loop(
                0, L, red,
                tuple(jnp.zeros((16,), jnp.float32) for _ in range(NJ)),
                unroll=2)
            b = ci * _G + g
            for j in range(NJ):
              sums_v[b, pl.ds(j * 16, 16)] = accs[j]
        return carry

      lax.fori_loop(0, NCHUNK // 2, outer, 0)
      pltpu.sync_copy(sums_v, out.at[pl.ds(base, BW)])

  return sc_kernel(name, collaborative, track_uri_can, *seq_flat, *tables)


def _tc_dense(embs, seq_idx, t0s, tail_raw, m_vec, inv_vec,
              U_p, V_p, b_p, W1_p, b1, W2, b2, W3, b3, L):
  B = embs[0].shape[0]
  ED = embs[0].shape[1]
  DP = U_p.shape[0]
  BB = 256
  grid = (B // BB,)

  def body(name_r, collab_r, can_r, an_sr, tu_sr, tn_sr, al_sr, ag_sr,
           an_ir, tu_ir, tn_ir, al_ir, ag_ir, t0_r, tail_r, m_r, inv_r,
           U_r, V_r, bc_r, W1_r, b1_r, W2_r, b2_r, W3_r, b3_r, out_r):
    adjs = []
    for f, (sr, ir) in enumerate(((an_sr, an_ir), (tu_sr, tu_ir),
                                  (tn_sr, tn_ir), (al_sr, al_ir),
                                  (ag_sr, ag_ir))):
      idx = ir[...]
      n0 = jnp.sum((idx == 0).astype(jnp.float32), axis=1, keepdims=True)
      cnt = jnp.maximum(np.float32(L) - n0, 1.0)
      t0 = t0_r[f, :][None, :]
      adjs.append((sr[...] - n0 * t0) / cnt)
    tail = (tail_r[...] - m_r[...]) * inv_r[...]
    x = jnp.concatenate(
        [name_r[...], collab_r[...], can_r[...]] + adjs + [tail], axis=1)
    xu = jnp.dot(x, U_r[...])
    f_ = jnp.dot(xu, V_r[...]) + bc_r[...]
    x1 = x * f_ + x
    h = jax.nn.relu(jnp.dot(x1, W1_r[...]) + b1_r[...])
    h = jax.nn.relu(jnp.dot(h, W2_r[...]) + b2_r[...])
    h = jnp.dot(h, W3_r[...]) + b3_r[...]
    sq = jnp.maximum(jnp.sum(h * h, axis=1, keepdims=True), 1e-12)
    out_r[...] = h * lax.rsqrt(sq)

  bs_emb = pl.BlockSpec((BB, ED), lambda i: (i, 0))
  bs_idx = pl.BlockSpec((BB, L), lambda i: (i, 0))
  const = lambda shape: pl.BlockSpec(shape, lambda i: tuple(0 for _ in shape))

  return pl.pallas_call(
      body,
      grid=grid,
      in_specs=[bs_emb] * 8 + [bs_idx] * 5 + [
          const(t0s.shape), pl.BlockSpec((BB, tail_raw.shape[1]), lambda i: (i, 0)),
          const(m_vec.shape), const(inv_vec.shape),
          const(U_p.shape), const(V_p.shape), const(b_p.shape),
          const(W1_p.shape), const(b1.shape), const(W2.shape),
          const(b2.shape), const(W3.shape), const(b3.shape),
      ],
      out_specs=pl.BlockSpec((BB, W3.shape[1]), lambda i: (i, 0)),
      out_shape=jax.ShapeDtypeStruct((B, W3.shape[1]), jnp.float32),
  )(*embs, *seq_idx, t0s, tail_raw, m_vec, inv_vec,
    U_p, V_p, b_p, W1_p, b1, W2, b2, W3, b3)


def kernel(name, collaborative, track_uri_can, n_songs_pl, num_artists_pl,
           num_albums_pl, artist_name_pl, track_uri_pl, track_name_pl,
           duration_ms_songs_pl, album_name_pl, artist_pop_pl,
           artists_followers_pl, track_pop_pl, artist_genres_pl,
           pl_name_table, collab_table, track_uri_can_table,
           artist_name_table, track_uri_pl_table, track_name_table,
           album_name_table, artist_genres_table,
           cross_U, cross_V, cross_b, W1, b1, W2, b2, W3, b3):
  B = name.shape[0]
  L = artist_name_pl.shape[1]
  ED = pl_name_table.shape[1]
  D_ALL = cross_U.shape[0]

  tables = [pl_name_table, collab_table, track_uri_can_table,
            artist_name_table, track_uri_pl_table, track_name_table,
            album_name_table, artist_genres_table]

  t0s = jnp.concatenate(
      [artist_name_table[0:1], track_uri_pl_table[0:1], track_name_table[0:1],
       album_name_table[0:1], artist_genres_table[0:1],
       jnp.zeros((3, ED), jnp.float32)], axis=0)

  n_tail = 3 + 4 * L
  tail_pad = 128 * ((n_tail + 127) // 128) - n_tail
  tail_raw = jnp.concatenate(
      [n_songs_pl[:, None], num_artists_pl[:, None], num_albums_pl[:, None],
       duration_ms_songs_pl, artist_pop_pl, artists_followers_pl,
       track_pop_pl, jnp.zeros((B, tail_pad), jnp.float32)], axis=1)

  m_np = np.concatenate([
      np.array([58.6523, 32.42, 42.73]),
      np.full(L, 234762.99476987208), np.full(L, 16.08),
      np.full(L, 7045512.193), np.full(L, 38.937), np.zeros(tail_pad)])
  v_np = np.concatenate([
      np.array([2275.8927, 763.0, 1290.0]),
      np.full(L, 5411028799.28701), np.full(L, 300.64),
      np.full(L, 178225831161684.75), np.full(L, 922.0), np.ones(tail_pad)])
  inv_np = 1.0 / np.sqrt(v_np)
  inv_np[n_tail:] = 0.0
  m_vec = jnp.asarray(m_np, jnp.float32)[None, :]
  inv_vec = jnp.asarray(inv_np, jnp.float32)[None, :]

  segs = {"name": (0, ED), "collab": (ED, ED), "can": (2 * ED, ED),
          "scal3": (3 * ED, 3), "an": (3 * ED + 3, ED),
          "tu": (4 * ED + 3, ED), "tn": (5 * ED + 3, ED),
          "dur": (6 * ED + 3, L), "al": (6 * ED + 3 + L, ED),
          "pop": (7 * ED + 3 + L, L), "fol": (7 * ED + 3 + 2 * L, L),
          "tpop": (7 * ED + 3 + 3 * L, L), "ag": (7 * ED + 3 + 4 * L, ED)}
  order = ["name", "collab", "can", "an", "tu", "tn", "al", "ag",
           "scal3", "dur", "pop", "fol", "tpop"]
  perm = np.concatenate([np.arange(segs[s][0], segs[s][0] + segs[s][1])
                         for s in order])
  DP = 8 * ED + (n_tail + tail_pad)
  PD = cross_U.shape[1]
  PDp = 128 * ((PD + 127) // 128)

  U_p = jnp.pad(cross_U[perm], ((0, DP - D_ALL), (0, PDp - PD)))
  V_p = jnp.pad(cross_V[:, perm], ((0, PDp - PD), (0, DP - D_ALL)))
  b_p = jnp.pad(cross_b[perm], (0, DP - D_ALL))[None, :]
  W1_p = jnp.pad(W1[perm], ((0, DP - D_ALL), (0, 0)))

  NSPLIT = 2
  H = B // NSPLIT
  seq_arrs = (artist_name_pl, track_uri_pl, track_name_pl, album_name_pl,
              artist_genres_pl)
  outs = []
  for s in range(NSPLIT):
    sl = slice(s * H, (s + 1) * H)
    seq_flat = [a[sl].reshape(-1) for a in seq_arrs]
    embs = _sc_gather_pool(name[sl], collaborative[sl], track_uri_can[sl],
                           seq_flat, tables)
    outs.append(_tc_dense(
        list(embs), [a[sl] for a in seq_arrs],
        t0s, tail_raw[sl], m_vec, inv_vec,
        U_p, V_p, b_p, W1_p, b1[None, :], W2, b2[None, :], W3,
        b3[None, :], L))
  return jnp.concatenate(outs, axis=0)


# speedup vs baseline: 1.0781x; 1.0781x over previous
import functools

import numpy as np
import jax
import jax.numpy as jnp
from jax import lax
from jax.experimental import pallas as pl
from jax.experimental.pallas import tpu as pltpu
from jax.experimental.pallas import tpu_sc as plsc

_NC = 2
_NS = 16
_NW = _NC * _NS
_G = 8


def _sc_gather_pool(name, collaborative, track_uri_can, seq_flat, tables):
  B = name.shape[0]
  L = seq_flat[0].shape[0] // B
  ED = tables[0].shape[1]
  BW = B // _NW
  NJ = ED // 16
  NCHUNK = BW // _G
  GL = _G * L

  mesh = plsc.VectorSubcoreMesh(
      core_axis_name="c", subcore_axis_name="s",
      num_cores=_NC, num_subcores=_NS)

  out_type = tuple(
      jax.ShapeDtypeStruct((B, ED), jnp.float32) for _ in range(8))

  @functools.partial(
      pl.kernel, mesh=mesh, out_type=out_type,
      scratch_types=[
          pltpu.VMEM((BW * L,), jnp.int32),
          pltpu.VMEM((GL, ED), jnp.float32),
          pltpu.VMEM((GL, ED), jnp.float32),
          pltpu.VMEM((BW, ED), jnp.float32),
          pltpu.SemaphoreType.DMA,
          pltpu.SemaphoreType.DMA,
          pltpu.SemaphoreType.DMA,
      ])
  def sc_kernel(name_h, collab_h, can_h, an_h, tu_h, tn_h, al_h, ag_h,
                t_name, t_collab, t_can, t_an, t_tu, t_tn, t_al, t_ag,
                o_name, o_collab, o_can, o_an, o_tu, o_tn, o_al, o_ag,
                sidx_v, buf0, buf1, sums_v,
                sem_a, sem0, sem1):
    rows_v = sums_v
    wid = lax.axis_index("s") * _NC + lax.axis_index("c")
    base = wid * BW

    singles = ((name_h, t_name, o_name), (collab_h, t_collab, o_collab),
               (can_h, t_can, o_can))
    sbufs = (rows_v, buf0.at[pl.ds(0, BW)], buf1.at[pl.ds(0, BW)])
    ssems = (sem_a, sem0, sem1)
    for i, (idx_h, tab, out) in enumerate(singles):
      pltpu.sync_copy(idx_h.at[pl.ds(base, BW)], sidx_v.at[pl.ds(i * BW, BW)])
    for i, (idx_h, tab, out) in enumerate(singles):
      pltpu.async_copy(tab.at[sidx_v.at[pl.ds(i * BW, BW)]], sbufs[i],
                       ssems[i])
    for i, (idx_h, tab, out) in enumerate(singles):
      pltpu.make_async_copy(tab.at[sidx_v.at[pl.ds(i * BW, BW)]], sbufs[i],
                            ssems[i]).wait()
      pltpu.sync_copy(sbufs[i], out.at[pl.ds(base, BW)])

    bufs = (buf0, buf1)
    sems = (sem0, sem1)
    for seq_h, tab, out in ((an_h, t_an, o_an), (tu_h, t_tu, o_tu),
                            (tn_h, t_tn, o_tn), (al_h, t_al, o_al),
                            (ag_h, t_ag, o_ag)):
      pltpu.sync_copy(seq_h.at[pl.ds(base * L, BW * L)], sidx_v)

      def start(ci, k):
        pltpu.async_copy(
            tab.at[sidx_v.at[pl.ds(ci * GL, GL)]], bufs[k], sems[k])

      def wait(k):
        pltpu.make_async_copy(
            tab.at[sidx_v.at[pl.ds(0, GL)]], bufs[k], sems[k]).wait()

      start(0, 0)
      start(1, 1)

      def outer(c2, carry):
        for k in range(2):
          ci = c2 * 2 + k
          wait(k)

          @pl.when(ci + 2 < NCHUNK)
          def _(k=k, ci=ci):
            start(ci + 2, k)

          buf = bufs[k]
          for g in range(_G):
            def red(l, accs, _g=g, _buf=buf):
              row = _g * L + l
              return tuple(accs[j] + _buf[row, pl.ds(j * 16, 16)]
                           for j in range(NJ))
            accs = lax.fori_loop(
                0, L, red,
                tuple(jnp.zeros((16,), jnp.float32) for _ in range(NJ)),
                unroll=2)
            b = ci * _G + g
            for j in range(NJ):
              sums_v[b, pl.ds(j * 16, 16)] = accs[j]
        return carry

      lax.fori_loop(0, NCHUNK // 2, outer, 0)
      pltpu.sync_copy(sums_v, out.at[pl.ds(base, BW)])

  return sc_kernel(name, collaborative, track_uri_can, *seq_flat, *tables)


def _tc_dense(embs, seq_idx, t0s, tail_raw, m_vec, inv_vec,
              U_p, V_p, b_p, W1_p, b1, W2, b2, W3, b3, L):
  B = embs[0].shape[0]
  ED = embs[0].shape[1]
  DP = U_p.shape[0]
  BB = 256
  grid = (B // BB,)

  def body(name_r, collab_r, can_r, an_sr, tu_sr, tn_sr, al_sr, ag_sr,
           an_ir, tu_ir, tn_ir, al_ir, ag_ir, t0_r, tail_r, m_r, inv_r,
           U_r, V_r, bc_r, W1_r, b1_r, W2_r, b2_r, W3_r, b3_r, out_r):
    adjs = []
    for f, (sr, ir) in enumerate(((an_sr, an_ir), (tu_sr, tu_ir),
                                  (tn_sr, tn_ir), (al_sr, al_ir),
                                  (ag_sr, ag_ir))):
      idx = ir[...]
      n0 = jnp.sum((idx == 0).astype(jnp.float32), axis=1, keepdims=True)
      cnt = jnp.maximum(np.float32(L) - n0, 1.0)
      t0 = t0_r[f, :][None, :]
      adjs.append((sr[...] - n0 * t0) / cnt)
    tail = (tail_r[...] - m_r[...]) * inv_r[...]
    x = jnp.concatenate(
        [name_r[...], collab_r[...], can_r[...]] + adjs + [tail], axis=1)
    xu = jnp.dot(x, U_r[...])
    f_ = jnp.dot(xu, V_r[...]) + bc_r[...]
    x1 = x * f_ + x
    h = jax.nn.relu(jnp.dot(x1, W1_r[...]) + b1_r[...])
    h = jax.nn.relu(jnp.dot(h, W2_r[...]) + b2_r[...])
    h = jnp.dot(h, W3_r[...]) + b3_r[...]
    sq = jnp.maximum(jnp.sum(h * h, axis=1, keepdims=True), 1e-12)
    out_r[...] = h * lax.rsqrt(sq)

  bs_emb = pl.BlockSpec((BB, ED), lambda i: (i, 0))
  bs_idx = pl.BlockSpec((BB, L), lambda i: (i, 0))
  const = lambda shape: pl.BlockSpec(shape, lambda i: tuple(0 for _ in shape))

  return pl.pallas_call(
      body,
      grid=grid,
      in_specs=[bs_emb] * 8 + [bs_idx] * 5 + [
          const(t0s.shape), pl.BlockSpec((BB, tail_raw.shape[1]), lambda i: (i, 0)),
          const(m_vec.shape), const(inv_vec.shape),
          const(U_p.shape), const(V_p.shape), const(b_p.shape),
          const(W1_p.shape), const(b1.shape), const(W2.shape),
          const(b2.shape), const(W3.shape), const(b3.shape),
      ],
      out_specs=pl.BlockSpec((BB, W3.shape[1]), lambda i: (i, 0)),
      out_shape=jax.ShapeDtypeStruct((B, W3.shape[1]), jnp.float32),
  )(*embs, *seq_idx, t0s, tail_raw, m_vec, inv_vec,
    U_p, V_p, b_p, W1_p, b1, W2, b2, W3, b3)


def kernel(name, collaborative, track_uri_can, n_songs_pl, num_artists_pl,
           num_albums_pl, artist_name_pl, track_uri_pl, track_name_pl,
           duration_ms_songs_pl, album_name_pl, artist_pop_pl,
           artists_followers_pl, track_pop_pl, artist_genres_pl,
           pl_name_table, collab_table, track_uri_can_table,
           artist_name_table, track_uri_pl_table, track_name_table,
           album_name_table, artist_genres_table,
           cross_U, cross_V, cross_b, W1, b1, W2, b2, W3, b3):
  B = name.shape[0]
  L = artist_name_pl.shape[1]
  ED = pl_name_table.shape[1]
  D_ALL = cross_U.shape[0]

  tables = [pl_name_table, collab_table, track_uri_can_table,
            artist_name_table, track_uri_pl_table, track_name_table,
            album_name_table, artist_genres_table]

  t0s = jnp.concatenate(
      [artist_name_table[0:1], track_uri_pl_table[0:1], track_name_table[0:1],
       album_name_table[0:1], artist_genres_table[0:1],
       jnp.zeros((3, ED), jnp.float32)], axis=0)

  n_tail = 3 + 4 * L
  tail_pad = 128 * ((n_tail + 127) // 128) - n_tail
  tail_raw = jnp.concatenate(
      [n_songs_pl[:, None], num_artists_pl[:, None], num_albums_pl[:, None],
       duration_ms_songs_pl, artist_pop_pl, artists_followers_pl,
       track_pop_pl, jnp.zeros((B, tail_pad), jnp.float32)], axis=1)

  m_np = np.concatenate([
      np.array([58.6523, 32.42, 42.73]),
      np.full(L, 234762.99476987208), np.full(L, 16.08),
      np.full(L, 7045512.193), np.full(L, 38.937), np.zeros(tail_pad)])
  v_np = np.concatenate([
      np.array([2275.8927, 763.0, 1290.0]),
      np.full(L, 5411028799.28701), np.full(L, 300.64),
      np.full(L, 178225831161684.75), np.full(L, 922.0), np.ones(tail_pad)])
  inv_np = 1.0 / np.sqrt(v_np)
  inv_np[n_tail:] = 0.0
  m_vec = jnp.asarray(m_np, jnp.float32)[None, :]
  inv_vec = jnp.asarray(inv_np, jnp.float32)[None, :]

  segs = {"name": (0, ED), "collab": (ED, ED), "can": (2 * ED, ED),
          "scal3": (3 * ED, 3), "an": (3 * ED + 3, ED),
          "tu": (4 * ED + 3, ED), "tn": (5 * ED + 3, ED),
          "dur": (6 * ED + 3, L), "al": (6 * ED + 3 + L, ED),
          "pop": (7 * ED + 3 + L, L), "fol": (7 * ED + 3 + 2 * L, L),
          "tpop": (7 * ED + 3 + 3 * L, L), "ag": (7 * ED + 3 + 4 * L, ED)}
  order = ["name", "collab", "can", "an", "tu", "tn", "al", "ag",
           "scal3", "dur", "pop", "fol", "tpop"]
  perm = np.concatenate([np.arange(segs[s][0], segs[s][0] + segs[s][1])
                         for s in order])
  DP = 8 * ED + (n_tail + tail_pad)
  PD = cross_U.shape[1]
  PDp = 128 * ((PD + 127) // 128)

  U_p = jnp.pad(cross_U[perm], ((0, DP - D_ALL), (0, PDp - PD)))
  V_p = jnp.pad(cross_V[:, perm], ((0, PDp - PD), (0, DP - D_ALL)))
  b_p = jnp.pad(cross_b[perm], (0, DP - D_ALL))[None, :]
  W1_p = jnp.pad(W1[perm], ((0, DP - D_ALL), (0, 0)))

  seq_arrs = (artist_name_pl, track_uri_pl, track_name_pl, album_name_pl,
              artist_genres_pl)
  seq_flat = [a.reshape(-1) for a in seq_arrs]
  embs = _sc_gather_pool(name, collaborative, track_uri_can, seq_flat,
                         tables)
  return _tc_dense(
      list(embs), list(seq_arrs),
      t0s, tail_raw, m_vec, inv_vec,
      U_p, V_p, b_p, W1_p, b1[None, :], W2, b2[None, :], W3,
      b3[None, :], L)


# speedup vs baseline: 1.1339x; 1.0518x over previous
import functools

import numpy as np
import jax
import jax.numpy as jnp
from jax import lax
from jax.experimental import pallas as pl
from jax.experimental.pallas import tpu as pltpu
from jax.experimental.pallas import tpu_sc as plsc

_NC = 2
_NS = 16
_NW = _NC * _NS
_G = 8


def _sc_gather_pool(name, collaborative, track_uri_can, seq_flat, tables):
  B = name.shape[0]
  L = seq_flat[0].shape[0] // B
  ED = tables[0].shape[1]
  BW = B // _NW
  NJ = ED // 16
  NCHUNK = BW // _G
  GL = _G * L

  mesh = plsc.VectorSubcoreMesh(
      core_axis_name="c", subcore_axis_name="s",
      num_cores=_NC, num_subcores=_NS)

  out_type = tuple(
      jax.ShapeDtypeStruct((B, ED), jnp.float32) for _ in range(8))

  @functools.partial(
      pl.kernel, mesh=mesh, out_type=out_type,
      scratch_types=[
          pltpu.VMEM((BW * L,), jnp.int32),
          pltpu.VMEM((2 * GL,), jnp.int32),
          pltpu.VMEM((2 * GL,), jnp.int32),
          pltpu.VMEM((GL, ED), jnp.float32),
          pltpu.VMEM((GL, ED), jnp.float32),
          pltpu.VMEM((BW, ED), jnp.float32),
          pltpu.SemaphoreType.DMA,
          pltpu.SemaphoreType.DMA,
          pltpu.SemaphoreType.DMA,
      ])
  def sc_kernel(name_h, collab_h, can_h, an_h, tu_h, tn_h, al_h, ag_h,
                t_name, t_collab, t_can, t_an, t_tu, t_tn, t_al, t_ag,
                o_name, o_collab, o_can, o_an, o_tu, o_tn, o_al, o_ag,
                sidx_v, pidx0, pidx1, buf0, buf1, sums_v,
                sem_a, sem0, sem1):
    rows_v = sums_v
    wid = lax.axis_index("s") * _NC + lax.axis_index("c")
    base = wid * BW

    singles = ((name_h, t_name, o_name), (collab_h, t_collab, o_collab),
               (can_h, t_can, o_can))
    sbufs = (rows_v, buf0.at[pl.ds(0, BW)], buf1.at[pl.ds(0, BW)])
    ssems = (sem_a, sem0, sem1)
    for i, (idx_h, tab, out) in enumerate(singles):
      pltpu.sync_copy(idx_h.at[pl.ds(base, BW)], sidx_v.at[pl.ds(i * BW, BW)])
    for i, (idx_h, tab, out) in enumerate(singles):
      pltpu.async_copy(tab.at[sidx_v.at[pl.ds(i * BW, BW)]], sbufs[i],
                       ssems[i])
    for i, (idx_h, tab, out) in enumerate(singles):
      pltpu.make_async_copy(tab.at[sidx_v.at[pl.ds(i * BW, BW)]], sbufs[i],
                            ssems[i]).wait()
      pltpu.sync_copy(sbufs[i], out.at[pl.ds(base, BW)])

    bufs = (buf0, buf1)
    sems = (sem0, sem1)
    pidxs = (pidx0, pidx1)
    feats = ((an_h, t_an, o_an), (tu_h, t_tu, o_tu), (tn_h, t_tn, o_tn),
             (al_h, t_al, o_al), (ag_h, t_ag, o_ag))
    NF = len(feats)

    def start(f, ci, k):
      pltpu.async_copy(
          feats[f][1].at[sidx_v.at[pl.ds(ci * GL, GL)]], bufs[k], sems[k])

    def start_pfx(f, ci, k):
      pltpu.async_copy(
          feats[f][1].at[pidxs[f % 2].at[pl.ds(ci * GL, GL)]],
          bufs[k], sems[k])

    def wait(f, k):
      pltpu.make_async_copy(
          feats[f][1].at[sidx_v.at[pl.ds(0, GL)]], bufs[k], sems[k]).wait()

    pltpu.sync_copy(feats[0][0].at[pl.ds(base * L, BW * L)], sidx_v)
    start(0, 0, 0)
    start(0, 1, 1)

    for f in range(NF):
      if f > 0:
        pltpu.sync_copy(feats[f][0].at[pl.ds(base * L, BW * L)], sidx_v)
      if f + 1 < NF:
        pltpu.sync_copy(feats[f + 1][0].at[pl.ds(base * L, 2 * GL)],
                        pidxs[(f + 1) % 2])
      out = feats[f][2]

      def outer(c2, carry, f=f):
        for k in range(2):
          ci = c2 * 2 + k
          wait(f, k)

          @pl.when(ci + 2 < NCHUNK)
          def _(k=k, ci=ci, f=f):
            start(f, ci + 2, k)

          if f + 1 < NF:
            @pl.when(ci + 2 >= NCHUNK)
            def _(k=k, ci=ci, f=f):
              start_pfx(f + 1, ci + 2 - NCHUNK, k)

          buf = bufs[k]
          for g in range(_G):
            def red(l, accs, _g=g, _buf=buf):
              row = _g * L + l
              return tuple(accs[j] + _buf[row, pl.ds(j * 16, 16)]
                           for j in range(NJ))
            accs = lax.fori_loop(
                0, L, red,
                tuple(jnp.zeros((16,), jnp.float32) for _ in range(NJ)),
                unroll=2)
            b = ci * _G + g
            for j in range(NJ):
              sums_v[b, pl.ds(j * 16, 16)] = accs[j]
        return carry

      lax.fori_loop(0, NCHUNK // 2, outer, 0)
      pltpu.sync_copy(sums_v, out.at[pl.ds(base, BW)])

  return sc_kernel(name, collaborative, track_uri_can, *seq_flat, *tables)


def _tc_dense(embs, seq_idx, t0s, tail_raw, m_vec, inv_vec,
              U_p, V_p, b_p, W1_p, b1, W2, b2, W3, b3, L):
  B = embs[0].shape[0]
  ED = embs[0].shape[1]
  DP = U_p.shape[0]
  BB = 256
  grid = (B // BB,)

  def body(name_r, collab_r, can_r, an_sr, tu_sr, tn_sr, al_sr, ag_sr,
           an_ir, tu_ir, tn_ir, al_ir, ag_ir, t0_r, tail_r, m_r, inv_r,
           U_r, V_r, bc_r, W1_r, b1_r, W2_r, b2_r, W3_r, b3_r, out_r):
    adjs = []
    for f, (sr, ir) in enumerate(((an_sr, an_ir), (tu_sr, tu_ir),
                                  (tn_sr, tn_ir), (al_sr, al_ir),
                                  (ag_sr, ag_ir))):
      idx = ir[...]
      n0 = jnp.sum((idx == 0).astype(jnp.float32), axis=1, keepdims=True)
      cnt = jnp.maximum(np.float32(L) - n0, 1.0)
      t0 = t0_r[f, :][None, :]
      adjs.append((sr[...] - n0 * t0) / cnt)
    tail = (tail_r[...] - m_r[...]) * inv_r[...]
    x = jnp.concatenate(
        [name_r[...], collab_r[...], can_r[...]] + adjs + [tail], axis=1)
    xu = jnp.dot(x, U_r[...])
    f_ = jnp.dot(xu, V_r[...]) + bc_r[...]
    x1 = x * f_ + x
    h = jax.nn.relu(jnp.dot(x1, W1_r[...]) + b1_r[...])
    h = jax.nn.relu(jnp.dot(h, W2_r[...]) + b2_r[...])
    h = jnp.dot(h, W3_r[...]) + b3_r[...]
    sq = jnp.maximum(jnp.sum(h * h, axis=1, keepdims=True), 1e-12)
    out_r[...] = h * lax.rsqrt(sq)

  bs_emb = pl.BlockSpec((BB, ED), lambda i: (i, 0))
  bs_idx = pl.BlockSpec((BB, L), lambda i: (i, 0))
  const = lambda shape: pl.BlockSpec(shape, lambda i: tuple(0 for _ in shape))

  return pl.pallas_call(
      body,
      grid=grid,
      in_specs=[bs_emb] * 8 + [bs_idx] * 5 + [
          const(t0s.shape), pl.BlockSpec((BB, tail_raw.shape[1]), lambda i: (i, 0)),
          const(m_vec.shape), const(inv_vec.shape),
          const(U_p.shape), const(V_p.shape), const(b_p.shape),
          const(W1_p.shape), const(b1.shape), const(W2.shape),
          const(b2.shape), const(W3.shape), const(b3.shape),
      ],
      out_specs=pl.BlockSpec((BB, W3.shape[1]), lambda i: (i, 0)),
      out_shape=jax.ShapeDtypeStruct((B, W3.shape[1]), jnp.float32),
  )(*embs, *seq_idx, t0s, tail_raw, m_vec, inv_vec,
    U_p, V_p, b_p, W1_p, b1, W2, b2, W3, b3)


def kernel(name, collaborative, track_uri_can, n_songs_pl, num_artists_pl,
           num_albums_pl, artist_name_pl, track_uri_pl, track_name_pl,
           duration_ms_songs_pl, album_name_pl, artist_pop_pl,
           artists_followers_pl, track_pop_pl, artist_genres_pl,
           pl_name_table, collab_table, track_uri_can_table,
           artist_name_table, track_uri_pl_table, track_name_table,
           album_name_table, artist_genres_table,
           cross_U, cross_V, cross_b, W1, b1, W2, b2, W3, b3):
  B = name.shape[0]
  L = artist_name_pl.shape[1]
  ED = pl_name_table.shape[1]
  D_ALL = cross_U.shape[0]

  tables = [pl_name_table, collab_table, track_uri_can_table,
            artist_name_table, track_uri_pl_table, track_name_table,
            album_name_table, artist_genres_table]

  t0s = jnp.concatenate(
      [artist_name_table[0:1], track_uri_pl_table[0:1], track_name_table[0:1],
       album_name_table[0:1], artist_genres_table[0:1],
       jnp.zeros((3, ED), jnp.float32)], axis=0)

  n_tail = 3 + 4 * L
  tail_pad = 128 * ((n_tail + 127) // 128) - n_tail
  tail_raw = jnp.concatenate(
      [n_songs_pl[:, None], num_artists_pl[:, None], num_albums_pl[:, None],
       duration_ms_songs_pl, artist_pop_pl, artists_followers_pl,
       track_pop_pl, jnp.zeros((B, tail_pad), jnp.float32)], axis=1)

  m_np = np.concatenate([
      np.array([58.6523, 32.42, 42.73]),
      np.full(L, 234762.99476987208), np.full(L, 16.08),
      np.full(L, 7045512.193), np.full(L, 38.937), np.zeros(tail_pad)])
  v_np = np.concatenate([
      np.array([2275.8927, 763.0, 1290.0]),
      np.full(L, 5411028799.28701), np.full(L, 300.64),
      np.full(L, 178225831161684.75), np.full(L, 922.0), np.ones(tail_pad)])
  inv_np = 1.0 / np.sqrt(v_np)
  inv_np[n_tail:] = 0.0
  m_vec = jnp.asarray(m_np, jnp.float32)[None, :]
  inv_vec = jnp.asarray(inv_np, jnp.float32)[None, :]

  segs = {"name": (0, ED), "collab": (ED, ED), "can": (2 * ED, ED),
          "scal3": (3 * ED, 3), "an": (3 * ED + 3, ED),
          "tu": (4 * ED + 3, ED), "tn": (5 * ED + 3, ED),
          "dur": (6 * ED + 3, L), "al": (6 * ED + 3 + L, ED),
          "pop": (7 * ED + 3 + L, L), "fol": (7 * ED + 3 + 2 * L, L),
          "tpop": (7 * ED + 3 + 3 * L, L), "ag": (7 * ED + 3 + 4 * L, ED)}
  order = ["name", "collab", "can", "an", "tu", "tn", "al", "ag",
           "scal3", "dur", "pop", "fol", "tpop"]
  perm = np.concatenate([np.arange(segs[s][0], segs[s][0] + segs[s][1])
                         for s in order])
  DP = 8 * ED + (n_tail + tail_pad)
  PD = cross_U.shape[1]
  PDp = 128 * ((PD + 127) // 128)

  U_p = jnp.pad(cross_U[perm], ((0, DP - D_ALL), (0, PDp - PD)))
  V_p = jnp.pad(cross_V[:, perm], ((0, PDp - PD), (0, DP - D_ALL)))
  b_p = jnp.pad(cross_b[perm], (0, DP - D_ALL))[None, :]
  W1_p = jnp.pad(W1[perm], ((0, DP - D_ALL), (0, 0)))

  seq_arrs = (artist_name_pl, track_uri_pl, track_name_pl, album_name_pl,
              artist_genres_pl)
  seq_flat = [a.reshape(-1) for a in seq_arrs]
  embs = _sc_gather_pool(name, collaborative, track_uri_can, seq_flat,
                         tables)
  return _tc_dense(
      list(embs), list(seq_arrs),
      t0s, tail_raw, m_vec, inv_vec,
      U_p, V_p, b_p, W1_p, b1[None, :], W2, b2[None, :], W3,
      b3[None, :], L)
